# Initial kernel scaffold; baseline (speedup 1.0000x reference)
import jax
import jax.numpy as jnp
from jax.experimental import pallas as pl


def kernel(x, codebook):
    raise NotImplementedError("write your pallas kernel here")


# speedup vs baseline: 3.2605x; 3.2605x over previous
import functools

import jax
import jax.numpy as jnp
from jax.experimental import pallas as pl


def _vq_body(x_ref, cbt_ref, out_ref, loss_ref, *, k_real, scale):
    n = pl.program_id(0)
    xb = x_ref[0]
    c_dim, _t_dim = xb.shape

    m = jnp.mean(xb, axis=1, keepdims=True)
    var = jnp.sum((xb - m) ** 2, axis=1, keepdims=True) / (xb.shape[1] - 1)
    inv_sx = 1.0 / (jnp.sqrt(var) + 1e-4)

    cbt = cbt_ref[...]
    cm = jnp.mean(cbt, axis=0, keepdims=True)
    cvar = jnp.sum((cbt - cm) ** 2, axis=0, keepdims=True) / (c_dim - 1)
    inv_sk = 1.0 / (jnp.sqrt(cvar) + 1e-4)

    xs = xb * inv_sx
    ys = cbt * inv_sk
    dists = jax.lax.dot_general(
        ys, xs, (((0,), (0,)), ((), ())),
        preferred_element_type=jnp.float32)

    iota0 = jax.lax.broadcasted_iota(jnp.int32, dists.shape, 0)
    d = jnp.where(iota0 < k_real, dists, -jnp.inf)
    mx = jnp.max(d, axis=0, keepdims=True)
    cand = jnp.where(d == mx, iota0, dists.shape[0])
    kmin = jnp.min(cand, axis=0, keepdims=True)
    onehot = (iota0 == kmin).astype(jnp.float32)

    q = jnp.dot(cbt, onehot, precision=jax.lax.Precision.HIGHEST,
                preferred_element_type=jnp.float32)
    out_ref[0] = q

    partial = jnp.sum((xs - q) ** 2)

    @pl.when(n == 0)
    def _():
        loss_ref[...] = jnp.zeros_like(loss_ref)

    loss_ref[...] += (partial * scale).reshape(1, 1)


def kernel(x, codebook):
    n_dim, c_dim, t_dim = x.shape
    k_dim = codebook.shape[0]
    kp = 128
    cbt = jnp.pad(codebook.T, ((0, 0), (0, kp - k_dim)))

    body = functools.partial(
        _vq_body, k_real=k_dim, scale=2.0 / (n_dim * c_dim * t_dim))
    quant, loss = pl.pallas_call(
        body,
        grid=(n_dim,),
        in_specs=[
            pl.BlockSpec((1, c_dim, t_dim), lambda n: (n, 0, 0)),
            pl.BlockSpec((c_dim, kp), lambda n: (0, 0)),
        ],
        out_specs=[
            pl.BlockSpec((1, c_dim, t_dim), lambda n: (n, 0, 0)),
            pl.BlockSpec((1, 1), lambda n: (0, 0)),
        ],
        out_shape=[
            jax.ShapeDtypeStruct((n_dim, c_dim, t_dim), jnp.float32),
            jax.ShapeDtypeStruct((1, 1), jnp.float32),
        ],
    )(x, cbt)
    return quant, loss[0, 0]


# speedup vs baseline: 5.1571x; 1.5817x over previous
import functools

import jax
import jax.numpy as jnp
from jax.experimental import pallas as pl


def _vq_body(x_ref, cbt_ref, out_ref, loss_ref, *, k_real, scale):
    n = pl.program_id(0)
    xb = x_ref[0]
    c_dim, _t_dim = xb.shape

    m = jnp.mean(xb, axis=1, keepdims=True)
    var = jnp.sum((xb - m) ** 2, axis=1, keepdims=True) / (xb.shape[1] - 1)
    inv_sx = 1.0 / (jnp.sqrt(var) + 1e-4)

    cbt = cbt_ref[...]
    cm = jnp.mean(cbt, axis=0, keepdims=True)
    cvar = jnp.sum((cbt - cm) ** 2, axis=0, keepdims=True) / (c_dim - 1)
    inv_sk = 1.0 / (jnp.sqrt(cvar) + 1e-4)

    xs = xb * inv_sx
    ys = cbt * inv_sk
    dists = jax.lax.dot_general(
        ys, xs, (((0,), (0,)), ((), ())),
        preferred_element_type=jnp.float32)

    iota0 = jax.lax.broadcasted_iota(jnp.int32, dists.shape, 0)
    d = jnp.where(iota0 < k_real, dists, -jnp.inf)
    mx = jnp.max(d, axis=0, keepdims=True)
    cand = jnp.where(d == mx, iota0, dists.shape[0])
    kmin = jnp.min(cand, axis=0, keepdims=True)
    onehot = (iota0 == kmin).astype(jnp.float32)

    oh_bf = onehot.astype(jnp.bfloat16)
    cbt_hi = cbt.astype(jnp.bfloat16)
    cbt_lo = (cbt - cbt_hi.astype(jnp.float32)).astype(jnp.bfloat16)
    q = (jnp.dot(cbt_hi, oh_bf, preferred_element_type=jnp.float32)
         + jnp.dot(cbt_lo, oh_bf, preferred_element_type=jnp.float32))
    out_ref[0] = q

    partial = jnp.sum((xs - q) ** 2)

    @pl.when(n == 0)
    def _():
        loss_ref[...] = jnp.zeros_like(loss_ref)

    loss_ref[...] += (partial * scale).reshape(1, 1)


def kernel(x, codebook):
    n_dim, c_dim, t_dim = x.shape
    k_dim = codebook.shape[0]
    kp = 128
    cbt = jnp.pad(codebook.T, ((0, 0), (0, kp - k_dim)))

    body = functools.partial(
        _vq_body, k_real=k_dim, scale=2.0 / (n_dim * c_dim * t_dim))
    quant, loss = pl.pallas_call(
        body,
        grid=(n_dim,),
        in_specs=[
            pl.BlockSpec((1, c_dim, t_dim), lambda n: (n, 0, 0)),
            pl.BlockSpec((c_dim, kp), lambda n: (0, 0)),
        ],
        out_specs=[
            pl.BlockSpec((1, c_dim, t_dim), lambda n: (n, 0, 0)),
            pl.BlockSpec((1, 1), lambda n: (0, 0)),
        ],
        out_shape=[
            jax.ShapeDtypeStruct((n_dim, c_dim, t_dim), jnp.float32),
            jax.ShapeDtypeStruct((1, 1), jnp.float32),
        ],
    )(x, cbt)
    return quant, loss[0, 0]


# speedup vs baseline: 6.7184x; 1.3027x over previous
import functools

import jax
import jax.numpy as jnp
from jax.experimental import pallas as pl


def _vq_body(x_ref, cbt_ref, out_ref, loss_ref, *, k_real, scale):
    n = pl.program_id(0)
    xb = x_ref[0]
    c_dim, _t_dim = xb.shape

    m = jnp.mean(xb, axis=1, keepdims=True)
    var = jnp.sum((xb - m) ** 2, axis=1, keepdims=True) / (xb.shape[1] - 1)
    inv_sx = 1.0 / (jnp.sqrt(var) + 1e-4)

    cbt = cbt_ref[...]
    cm = jnp.mean(cbt, axis=0, keepdims=True)
    cvar = jnp.sum((cbt - cm) ** 2, axis=0, keepdims=True) / (c_dim - 1)
    inv_sk = 1.0 / (jnp.sqrt(cvar) + 1e-4)

    xs = xb * inv_sx
    ys = cbt * inv_sk
    dists = jax.lax.dot_general(
        ys, xs, (((0,), (0,)), ((), ())),
        preferred_element_type=jnp.float32)

    iota0 = jax.lax.broadcasted_iota(jnp.int32, dists.shape, 0)
    d = jnp.where(iota0 < k_real, dists, -jnp.inf)
    mx = jnp.max(d, axis=0, keepdims=True)
    cand = jnp.where(d == mx, iota0, dists.shape[0])
    kmin = jnp.min(cand, axis=0, keepdims=True)
    onehot = (iota0 == kmin).astype(jnp.float32)

    oh_bf = onehot.astype(jnp.bfloat16)
    q = jnp.dot(cbt.astype(jnp.bfloat16), oh_bf,
                preferred_element_type=jnp.float32)
    out_ref[0] = q

    t_dim = xb.shape[1]
    sum_xs2 = jnp.sum((inv_sx * inv_sx) * (var * (t_dim - 1) + t_dim * m * m))
    sel = onehot * dists
    row_xsq = jnp.sum(sel, axis=1, keepdims=True)
    sk_plus = jnp.sqrt(cvar) + 1e-4
    sum_xsq = jnp.dot(sk_plus, row_xsq,
                      preferred_element_type=jnp.float32)[0, 0]
    counts = jnp.sum(onehot, axis=1, keepdims=True)
    rownorm2 = jnp.sum(cbt * cbt, axis=0, keepdims=True)
    sum_q2 = jnp.dot(rownorm2, counts,
                     preferred_element_type=jnp.float32)[0, 0]
    partial = sum_xs2 - 2.0 * sum_xsq + sum_q2

    @pl.when(n == 0)
    def _():
        loss_ref[...] = jnp.zeros_like(loss_ref)

    loss_ref[...] += (partial * scale).reshape(1, 1)


def kernel(x, codebook):
    n_dim, c_dim, t_dim = x.shape
    k_dim = codebook.shape[0]
    kp = 128
    cbt = jnp.pad(codebook.T, ((0, 0), (0, kp - k_dim)))

    body = functools.partial(
        _vq_body, k_real=k_dim, scale=2.0 / (n_dim * c_dim * t_dim))
    quant, loss = pl.pallas_call(
        body,
        grid=(n_dim,),
        in_specs=[
            pl.BlockSpec((1, c_dim, t_dim), lambda n: (n, 0, 0)),
            pl.BlockSpec((c_dim, kp), lambda n: (0, 0)),
        ],
        out_specs=[
            pl.BlockSpec((1, c_dim, t_dim), lambda n: (n, 0, 0)),
            pl.BlockSpec((1, 1), lambda n: (0, 0)),
        ],
        out_shape=[
            jax.ShapeDtypeStruct((n_dim, c_dim, t_dim), jnp.float32),
            jax.ShapeDtypeStruct((1, 1), jnp.float32),
        ],
    )(x, cbt)
    return quant, loss[0, 0]


# speedup vs baseline: 6.7972x; 1.0117x over previous
import functools

import jax
import jax.numpy as jnp
from jax.experimental import pallas as pl


def _vq_body(x_ref, cbt_ref, out_ref, loss_ref, *, scale):
    n = pl.program_id(0)
    xb = x_ref[0]
    c_dim, t_dim = xb.shape

    m = jnp.mean(xb, axis=1, keepdims=True)
    var = jnp.sum((xb - m) ** 2, axis=1, keepdims=True) / (t_dim - 1)
    inv_sx = 1.0 / (jnp.sqrt(var) + 1e-4)

    cbt = cbt_ref[...]
    cm = jnp.mean(cbt, axis=0, keepdims=True)
    cvar = jnp.sum((cbt - cm) ** 2, axis=0, keepdims=True) / (c_dim - 1)
    sk_plus = jnp.sqrt(cvar) + 1e-4

    xs_bf = (xb * inv_sx).astype(jnp.bfloat16)
    ys_bf = (cbt / sk_plus).astype(jnp.bfloat16)
    dists = jax.lax.dot_general(
        ys_bf, xs_bf, (((0,), (0,)), ((), ())),
        preferred_element_type=jnp.float32)

    iota0 = jax.lax.broadcasted_iota(jnp.int32, dists.shape, 0)
    mx = jnp.max(dists, axis=0, keepdims=True)
    cand = jnp.where(dists == mx, iota0, dists.shape[0])
    kmin = jnp.min(cand, axis=0, keepdims=True)
    onehot = (iota0 == kmin).astype(jnp.float32)

    q = jnp.dot(cbt.astype(jnp.bfloat16), onehot.astype(jnp.bfloat16),
                preferred_element_type=jnp.float32)
    out_ref[0] = q

    sum_xs2 = jnp.sum((inv_sx * inv_sx) * (var * (t_dim - 1) + t_dim * m * m))
    row_xsq = jnp.sum(onehot * dists, axis=1, keepdims=True)
    sum_xsq = jnp.dot(sk_plus, row_xsq,
                      preferred_element_type=jnp.float32)[0, 0]
    counts = jnp.sum(onehot, axis=1, keepdims=True)
    rownorm2 = jnp.sum(cbt * cbt, axis=0, keepdims=True)
    sum_q2 = jnp.dot(rownorm2, counts,
                     preferred_element_type=jnp.float32)[0, 0]
    partial = sum_xs2 - 2.0 * sum_xsq + sum_q2

    @pl.when(n == 0)
    def _():
        loss_ref[...] = jnp.zeros_like(loss_ref)

    loss_ref[...] += (partial * scale).reshape(1, 1)


def kernel(x, codebook):
    n_dim, c_dim, t_dim = x.shape
    k_dim = codebook.shape[0]
    cbt = codebook.T

    body = functools.partial(_vq_body, scale=2.0 / (n_dim * c_dim * t_dim))
    quant, loss = pl.pallas_call(
        body,
        grid=(n_dim,),
        in_specs=[
            pl.BlockSpec((1, c_dim, t_dim), lambda n: (n, 0, 0)),
            pl.BlockSpec((c_dim, k_dim), lambda n: (0, 0)),
        ],
        out_specs=[
            pl.BlockSpec((1, c_dim, t_dim), lambda n: (n, 0, 0)),
            pl.BlockSpec((1, 1), lambda n: (0, 0)),
        ],
        out_shape=[
            jax.ShapeDtypeStruct((n_dim, c_dim, t_dim), jnp.float32),
            jax.ShapeDtypeStruct((1, 1), jnp.float32),
        ],
    )(x, cbt)
    return quant, loss[0, 0]


# speedup vs baseline: 6.9040x; 1.0157x over previous
import functools

import jax
import jax.numpy as jnp
from jax.experimental import pallas as pl


def _vq_body(x_ref, cbt_ref, out_ref, loss_ref, *, scale):
    n = pl.program_id(0)
    xb = x_ref[0]
    c_dim, t_dim = xb.shape

    m = jnp.mean(xb, axis=1, keepdims=True)
    sumsq = jnp.sum(xb * xb, axis=1, keepdims=True)
    var = (sumsq - t_dim * (m * m)) / (t_dim - 1)
    inv_sx = 1.0 / (jnp.sqrt(var) + 1e-4)

    cbt = cbt_ref[...]
    cm = jnp.mean(cbt, axis=0, keepdims=True)
    cvar = jnp.sum((cbt - cm) ** 2, axis=0, keepdims=True) / (c_dim - 1)
    sk_plus = jnp.sqrt(cvar) + 1e-4

    xs_bf = (xb * inv_sx).astype(jnp.bfloat16)
    ys_bf = (cbt / sk_plus).astype(jnp.bfloat16)
    dists = jax.lax.dot_general(
        ys_bf, xs_bf, (((0,), (0,)), ((), ())),
        preferred_element_type=jnp.float32)

    iota0 = jax.lax.broadcasted_iota(jnp.int32, dists.shape, 0)
    mx = jnp.max(dists, axis=0, keepdims=True)
    cand = jnp.where(dists == mx, iota0, dists.shape[0])
    kmin = jnp.min(cand, axis=0, keepdims=True)
    onehot = (iota0 == kmin).astype(jnp.float32)

    q = jnp.dot(cbt.astype(jnp.bfloat16), onehot.astype(jnp.bfloat16),
                preferred_element_type=jnp.float32)
    out_ref[0] = q

    sum_xs2 = jnp.sum((inv_sx * inv_sx) * sumsq)
    row_xsq = jnp.sum(onehot * dists, axis=1, keepdims=True)
    sum_xsq = jnp.dot(sk_plus, row_xsq,
                      preferred_element_type=jnp.float32)[0, 0]
    counts = jnp.sum(onehot, axis=1, keepdims=True)
    rownorm2 = jnp.sum(cbt * cbt, axis=0, keepdims=True)
    sum_q2 = jnp.dot(rownorm2, counts,
                     preferred_element_type=jnp.float32)[0, 0]
    partial = sum_xs2 - 2.0 * sum_xsq + sum_q2

    @pl.when(n == 0)
    def _():
        loss_ref[...] = jnp.zeros_like(loss_ref)

    loss_ref[...] += (partial * scale).reshape(1, 1)


def kernel(x, codebook):
    n_dim, c_dim, t_dim = x.shape
    k_dim = codebook.shape[0]
    cbt = codebook.T

    body = functools.partial(_vq_body, scale=2.0 / (n_dim * c_dim * t_dim))
    quant, loss = pl.pallas_call(
        body,
        grid=(n_dim,),
        in_specs=[
            pl.BlockSpec((1, c_dim, t_dim), lambda n: (n, 0, 0)),
            pl.BlockSpec((c_dim, k_dim), lambda n: (0, 0)),
        ],
        out_specs=[
            pl.BlockSpec((1, c_dim, t_dim), lambda n: (n, 0, 0)),
            pl.BlockSpec((1, 1), lambda n: (0, 0)),
        ],
        out_shape=[
            jax.ShapeDtypeStruct((n_dim, c_dim, t_dim), jnp.float32),
            jax.ShapeDtypeStruct((1, 1), jnp.float32),
        ],
    )(x, cbt)
    return quant, loss[0, 0]
